# Initial kernel scaffold; baseline (speedup 1.0000x reference)
import jax
import jax.numpy as jnp
from jax.experimental import pallas as pl


def kernel(hidden_states, gate_w, w1, w3, w2, ws1, ws3, ws2):
    raise NotImplementedError("write your pallas kernel here")


# speedup vs baseline: 1.9868x; 1.9868x over previous
import jax
import jax.numpy as jnp
from jax.experimental import pallas as pl
from jax.experimental.pallas import tpu as pltpu

_DC = 512


def _nt(a, b):
    return jax.lax.dot_general(a, b, (((1,), (1,)), ((), ())),
                               preferred_element_type=jnp.float32)


def _routed_body(x32_ref, gw_ref, w1_ref, w3_ref, w2_ref, final_ref,
                 xb_ref, p_ref):
    e = pl.program_id(0)
    c = pl.program_id(1)
    n_exp = gw_ref.shape[0]
    first = (e == 0) & (c == 0)

    @pl.when(first)
    def _():
        xb_ref[...] = x32_ref[...].astype(jnp.bfloat16)
        logits = _nt(x32_ref[...], gw_ref[...])
        p = jax.nn.sigmoid(logits)
        p = p / jnp.sum(p, axis=-1, keepdims=True)
        p = p / jnp.sum(p, axis=-1, keepdims=True)
        p_ref[...] = p

    xb = xb_ref[...]
    h1 = _nt(xb, w1_ref[0].astype(jnp.bfloat16))
    h3 = _nt(xb, w3_ref[0].astype(jnp.bfloat16))
    g = (h1 * jax.nn.sigmoid(h1)) * h3
    y = jax.lax.dot_general(g.astype(jnp.bfloat16),
                            w2_ref[0].astype(jnp.bfloat16),
                            (((1,), (1,)), ((), ())),
                            preferred_element_type=jnp.float32)

    mask = (jax.lax.broadcasted_iota(jnp.int32, (1, n_exp), 1) == e
            ).astype(jnp.float32)
    w_col = jnp.sum(p_ref[...] * mask, axis=-1, keepdims=True)
    contrib = w_col * y

    @pl.when(first)
    def _():
        final_ref[...] = contrib

    @pl.when(jnp.logical_not(first))
    def _():
        final_ref[...] += contrib


def _shared_body(x32_ref, ws1_ref, ws3_ref, ws2_ref, out_ref, xb_ref):
    c = pl.program_id(0)

    @pl.when(c == 0)
    def _():
        xb_ref[...] = x32_ref[...].astype(jnp.bfloat16)

    xb = xb_ref[...]
    h1 = _nt(xb, ws1_ref[...].astype(jnp.bfloat16))
    h3 = _nt(xb, ws3_ref[...].astype(jnp.bfloat16))
    g = (h1 * jax.nn.sigmoid(h1)) * h3
    y = jax.lax.dot_general(g.astype(jnp.bfloat16),
                            ws2_ref[...].astype(jnp.bfloat16),
                            (((1,), (1,)), ((), ())),
                            preferred_element_type=jnp.float32)

    @pl.when(c == 0)
    def _():
        out_ref[...] = y

    @pl.when(c > 0)
    def _():
        out_ref[...] += y


def kernel(hidden_states, gate_w, w1, w3, w2, ws1, ws3, ws2):
    orig_shape = hidden_states.shape
    d = orig_shape[-1]
    x = hidden_states.reshape(-1, d)
    t = x.shape[0]
    n_exp, dff = w1.shape[0], w1.shape[1]
    sff = ws1.shape[0]
    nc = dff // _DC
    nsc = sff // _DC

    final = pl.pallas_call(
        _routed_body,
        grid=(n_exp, nc),
        in_specs=[
            pl.BlockSpec((t, d), lambda e, c: (0, 0)),
            pl.BlockSpec((n_exp, d), lambda e, c: (0, 0)),
            pl.BlockSpec((1, _DC, d), lambda e, c: (e, c, 0)),
            pl.BlockSpec((1, _DC, d), lambda e, c: (e, c, 0)),
            pl.BlockSpec((1, d, _DC), lambda e, c: (e, 0, c)),
        ],
        out_specs=pl.BlockSpec((t, d), lambda e, c: (0, 0)),
        out_shape=jax.ShapeDtypeStruct((t, d), jnp.float32),
        scratch_shapes=[
            pltpu.VMEM((t, d), jnp.bfloat16),
            pltpu.VMEM((t, n_exp), jnp.float32),
        ],
        compiler_params=pltpu.CompilerParams(
            dimension_semantics=("arbitrary", "arbitrary"),
        ),
    )(x, gate_w, w1, w3, w2)

    shared_out = pl.pallas_call(
        _shared_body,
        grid=(nsc,),
        in_specs=[
            pl.BlockSpec((t, d), lambda c: (0, 0)),
            pl.BlockSpec((_DC, d), lambda c: (c, 0)),
            pl.BlockSpec((_DC, d), lambda c: (c, 0)),
            pl.BlockSpec((d, _DC), lambda c: (0, c)),
        ],
        out_specs=pl.BlockSpec((t, d), lambda c: (0, 0)),
        out_shape=jax.ShapeDtypeStruct((t, d), jnp.float32),
        scratch_shapes=[pltpu.VMEM((t, d), jnp.bfloat16)],
        compiler_params=pltpu.CompilerParams(
            dimension_semantics=("arbitrary",),
        ),
    )(x, ws1, ws3, ws2)

    return shared_out, final.reshape(orig_shape)


# speedup vs baseline: 2.0050x; 1.0092x over previous
import jax
import jax.numpy as jnp
from jax.experimental import pallas as pl
from jax.experimental.pallas import tpu as pltpu

_DC = 256
_SC = 512
_EPB = 2


def _nt(a, b):
    return jax.lax.dot_general(a, b, (((1,), (1,)), ((), ())),
                               preferred_element_type=jnp.float32)


def _routed_body(x32_ref, gw_ref, w1_ref, w3_ref, w2_ref, final_ref,
                 xb_ref, p_ref):
    pe = pl.program_id(0)
    c = pl.program_id(1)
    n_exp = gw_ref.shape[0]
    first = (pe == 0) & (c == 0)

    @pl.when(first)
    def _():
        xb_ref[...] = x32_ref[...].astype(jnp.bfloat16)
        logits = _nt(x32_ref[...], gw_ref[...])
        p = jax.nn.sigmoid(logits)
        p = p / jnp.sum(p, axis=-1, keepdims=True)
        p = p / jnp.sum(p, axis=-1, keepdims=True)
        p_ref[...] = p

    xb = xb_ref[...]
    p_all = p_ref[...]
    iota = jax.lax.broadcasted_iota(jnp.int32, (1, n_exp), 1)

    ys = []
    for i in range(_EPB):
        e = pe * _EPB + i
        h1 = _nt(xb, w1_ref[i].astype(jnp.bfloat16))
        h3 = _nt(xb, w3_ref[i].astype(jnp.bfloat16))
        g = (h1 * jax.nn.sigmoid(h1)) * h3
        w_col = jnp.sum(p_all * (iota == e).astype(jnp.float32),
                        axis=-1, keepdims=True)
        pg = (w_col * g).astype(jnp.bfloat16)
        ys.append(jax.lax.dot_general(pg, w2_ref[i].astype(jnp.bfloat16),
                                      (((1,), (1,)), ((), ())),
                                      preferred_element_type=jnp.float32))

    contrib = ys[0]
    for y in ys[1:]:
        contrib = contrib + y

    @pl.when(first)
    def _():
        final_ref[...] = contrib

    @pl.when(jnp.logical_not(first))
    def _():
        final_ref[...] += contrib


def _shared_body(x32_ref, ws1_ref, ws3_ref, ws2_ref, out_ref, xb_ref):
    c = pl.program_id(0)

    @pl.when(c == 0)
    def _():
        xb_ref[...] = x32_ref[...].astype(jnp.bfloat16)

    xb = xb_ref[...]
    h1 = _nt(xb, ws1_ref[...].astype(jnp.bfloat16))
    h3 = _nt(xb, ws3_ref[...].astype(jnp.bfloat16))
    g = (h1 * jax.nn.sigmoid(h1)) * h3
    y = jax.lax.dot_general(g.astype(jnp.bfloat16),
                            ws2_ref[...].astype(jnp.bfloat16),
                            (((1,), (1,)), ((), ())),
                            preferred_element_type=jnp.float32)

    @pl.when(c == 0)
    def _():
        out_ref[...] = y

    @pl.when(c > 0)
    def _():
        out_ref[...] += y


def kernel(hidden_states, gate_w, w1, w3, w2, ws1, ws3, ws2):
    orig_shape = hidden_states.shape
    d = orig_shape[-1]
    x = hidden_states.reshape(-1, d)
    t = x.shape[0]
    n_exp, dff = w1.shape[0], w1.shape[1]
    sff = ws1.shape[0]
    nc = dff // _DC
    nsc = sff // _SC

    final = pl.pallas_call(
        _routed_body,
        grid=(n_exp // _EPB, nc),
        in_specs=[
            pl.BlockSpec((t, d), lambda e, c: (0, 0)),
            pl.BlockSpec((n_exp, d), lambda e, c: (0, 0)),
            pl.BlockSpec((_EPB, _DC, d), lambda e, c: (e, c, 0)),
            pl.BlockSpec((_EPB, _DC, d), lambda e, c: (e, c, 0)),
            pl.BlockSpec((_EPB, d, _DC), lambda e, c: (e, 0, c)),
        ],
        out_specs=pl.BlockSpec((t, d), lambda e, c: (0, 0)),
        out_shape=jax.ShapeDtypeStruct((t, d), jnp.float32),
        scratch_shapes=[
            pltpu.VMEM((t, d), jnp.bfloat16),
            pltpu.VMEM((t, n_exp), jnp.float32),
        ],
        compiler_params=pltpu.CompilerParams(
            dimension_semantics=("arbitrary", "arbitrary"),
        ),
    )(x, gate_w, w1, w3, w2)

    shared_out = pl.pallas_call(
        _shared_body,
        grid=(nsc,),
        in_specs=[
            pl.BlockSpec((t, d), lambda c: (0, 0)),
            pl.BlockSpec((_SC, d), lambda c: (c, 0)),
            pl.BlockSpec((_SC, d), lambda c: (c, 0)),
            pl.BlockSpec((d, _SC), lambda c: (0, c)),
        ],
        out_specs=pl.BlockSpec((t, d), lambda c: (0, 0)),
        out_shape=jax.ShapeDtypeStruct((t, d), jnp.float32),
        scratch_shapes=[pltpu.VMEM((t, d), jnp.bfloat16)],
        compiler_params=pltpu.CompilerParams(
            dimension_semantics=("arbitrary",),
        ),
    )(x, ws1, ws3, ws2)

    return shared_out, final.reshape(orig_shape)
